# Initial kernel scaffold; baseline (speedup 1.0000x reference)
import jax
import jax.numpy as jnp
from jax.experimental import pallas as pl


def kernel(z, point_num, sphere, W1, b1, W2, b2, W3, b3, W4, b4, gamma, beta, Wx1, bx1, Wx2, bx2):
    raise NotImplementedError("write your pallas kernel here")


# speedup vs baseline: 1313.9702x; 1313.9702x over previous
import functools
import jax
import jax.numpy as jnp
from jax.experimental import pallas as pl
from jax.experimental.pallas import tpu as pltpu

B = 4
N = 2048
K = 8
C = 512


def _leaky(x, a):
    return jnp.maximum(x, a * x)


def _knn_body(x_ref, xt_ref, sqc_ref, sqr_ref, idx_ref, *, n, k):
    b = pl.program_id(0)
    xr = x_ref[0]
    xt = xt_ref[0]
    sq_r = sqc_ref[0]
    sq_all = sqr_ref[0]
    g = jnp.dot(xr, xt, preferred_element_type=jnp.float32)
    d = sq_r + sq_all - 2.0 * g
    neg = -d
    r = xr.shape[0]
    iota = jax.lax.broadcasted_iota(jnp.int32, (r, n), 1)
    cols = []
    for _ in range(k):
        m = jnp.max(neg, axis=1, keepdims=True)
        cand = jnp.where(neg == m, iota, n)
        sel = jnp.min(cand, axis=1, keepdims=True)
        cols.append(sel)
        neg = jnp.where(iota == sel, -jnp.inf, neg)
    idx_ref[0] = jnp.concatenate(cols, axis=1) + b * n


def _knn(x, sq):
    b, n, c = x.shape
    xt = jnp.swapaxes(x, 1, 2)
    sq_col = sq[:, :, None]
    sq_row = sq[:, None, :]
    r = 256
    return pl.pallas_call(
        functools.partial(_knn_body, n=n, k=K),
        grid=(b, n // r),
        in_specs=[
            pl.BlockSpec((1, r, c), lambda bi, i: (bi, i, 0)),
            pl.BlockSpec((1, c, n), lambda bi, i: (bi, 0, 0)),
            pl.BlockSpec((1, r, 1), lambda bi, i: (bi, i, 0)),
            pl.BlockSpec((1, 1, n), lambda bi, i: (bi, 0, 0)),
        ],
        out_specs=pl.BlockSpec((1, r, K), lambda bi, i: (bi, i, 0)),
        out_shape=jax.ShapeDtypeStruct((b, n, K), jnp.int32),
    )(x, xt, sq_col, sq_row)


def _linear_body(x_ref, wt_ref, b_ref, o_ref, *, act):
    t = jnp.dot(x_ref[0], wt_ref[...], preferred_element_type=jnp.float32)
    t = t + b_ref[...]
    if act == "leaky":
        t = _leaky(t, 0.2)
    o_ref[0] = t


def _linear(x, wt, bias, act):
    b, n, c = x.shape
    o = wt.shape[1]
    r = 512
    return pl.pallas_call(
        functools.partial(_linear_body, act=act),
        grid=(b, n // r),
        in_specs=[
            pl.BlockSpec((1, r, c), lambda bi, i: (bi, i, 0)),
            pl.BlockSpec((c, o), lambda bi, i: (0, 0)),
            pl.BlockSpec((1, o), lambda bi, i: (0, 0)),
        ],
        out_specs=pl.BlockSpec((1, r, o), lambda bi, i: (bi, i, 0)),
        out_shape=jax.ShapeDtypeStruct((b, n, o), jnp.float32),
    )(x, wt, bias)


def _gmean_body(idx_ref, tab_ref, o_ref, *, n):
    b = pl.program_id(0)
    idx = idx_ref[0]
    r = idx.shape[0]
    iota = jax.lax.broadcasted_iota(jnp.int32, (r, n), 1) + b * n
    m = jnp.zeros((r, n), jnp.float32)
    for k in range(K):
        m = m + (iota == idx[:, k:k + 1]).astype(jnp.float32)
    o_ref[0] = jnp.dot(m, tab_ref[0], preferred_element_type=jnp.float32,
                       precision=jax.lax.Precision.HIGHEST) * (1.0 / K)


def _gather_mean(table, idx):
    b, n, c = table.shape
    r = 256
    return pl.pallas_call(
        functools.partial(_gmean_body, n=n),
        grid=(b, n // r),
        in_specs=[
            pl.BlockSpec((1, r, K), lambda bi, i: (bi, i, 0)),
            pl.BlockSpec((1, n, c), lambda bi, i: (bi, 0, 0)),
        ],
        out_specs=pl.BlockSpec((1, r, c), lambda bi, i: (bi, i, 0)),
        out_shape=jax.ShapeDtypeStruct((b, n, c), jnp.float32),
    )(idx, table)


def _layer1_body(idx_ref, s_ref, ws_ref, wg_ref, zc_ref, o_ref, *, n):
    i = pl.program_id(0)
    r = idx_ref.shape[0]
    idx = idx_ref[...]
    s_all = s_ref[...]
    s_blk = s_ref[pl.ds(i * r, r), :]
    a2 = jnp.dot(s_blk, ws_ref[...], preferred_element_type=jnp.float32)
    iota = jax.lax.broadcasted_iota(jnp.int32, (r, n), 1)
    accs = [jnp.zeros((r, C), jnp.float32) for _ in range(B)]
    for k in range(K):
        mk = (iota == idx[:, k:k + 1]).astype(jnp.float32)
        gx = jnp.dot(mk, s_all, preferred_element_type=jnp.float32,
                     precision=jax.lax.Precision.HIGHEST)
        dk = jnp.dot(gx - s_blk, wg_ref[...],
                     preferred_element_type=jnp.float32)
        base = a2 + dk
        for bi in range(B):
            accs[bi] = accs[bi] + _leaky(base + zc_ref[bi:bi + 1, :], 0.2)
    for bi in range(B):
        o_ref[bi] = accs[bi] * (1.0 / K)


def _layer1(idx0, s_pad, wst, wgt, zc):
    n = s_pad.shape[0]
    r = 256
    return pl.pallas_call(
        functools.partial(_layer1_body, n=n),
        grid=(n // r,),
        in_specs=[
            pl.BlockSpec((r, K), lambda i: (i, 0)),
            pl.BlockSpec((n, 8), lambda i: (0, 0)),
            pl.BlockSpec((8, C), lambda i: (0, 0)),
            pl.BlockSpec((8, C), lambda i: (0, 0)),
            pl.BlockSpec((B, C), lambda i: (0, 0)),
        ],
        out_specs=pl.BlockSpec((B, r, C), lambda i: (0, i, 0)),
        out_shape=jax.ShapeDtypeStruct((B, n, C), jnp.float32),
    )(idx0, s_pad, wst, wgt, zc)


def _prep1_body(z_ref, wz_ref, b1_ref, zc_ref):
    zc_ref[...] = jnp.dot(z_ref[...], wz_ref[...],
                          preferred_element_type=jnp.float32) + b1_ref[...]


def _prep1(z, wz, b1):
    return pl.pallas_call(
        _prep1_body,
        out_shape=jax.ShapeDtypeStruct((B, C), jnp.float32),
    )(z, wz, b1)


def _stats_body(idx_ref, q_ref, gm_ref, bt_ref, ac_ref, s_ref, *, n):
    b = pl.program_id(0)
    idx = idx_ref[0]
    q = q_ref[0]
    cnt = jnp.zeros((1, n), jnp.float32)
    r = 256
    for i in range(n // r):
        blk = idx[i * r:(i + 1) * r]
        iota = jax.lax.broadcasted_iota(jnp.int32, (r, n), 1) + b * n
        m = jnp.zeros((r, n), jnp.float32)
        for k in range(K):
            m = m + (iota == blk[:, k:k + 1]).astype(jnp.float32)
        cnt = cnt + jnp.sum(m, axis=0, keepdims=True)
    s1 = jnp.dot(cnt, q, preferred_element_type=jnp.float32,
                 precision=jax.lax.Precision.HIGHEST)
    s2 = jnp.dot(cnt, q * q, preferred_element_type=jnp.float32,
                 precision=jax.lax.Precision.HIGHEST)

    @pl.when(b == 0)
    def _():
        s_ref[...] = jnp.zeros_like(s_ref)

    s_ref[0:1, :] += s1
    s_ref[1:2, :] += s2

    @pl.when(b == pl.num_programs(0) - 1)
    def _():
        denom = 1.0 / (B * n * K)
        mu = s_ref[0:1, :] * denom
        var = s_ref[1:2, :] * denom - mu * mu
        a = gm_ref[...] / jnp.sqrt(var + 1e-5)
        c = bt_ref[...] - mu * a
        ac_ref[...] = jnp.concatenate([a, c], axis=0)


def _stats(idx4, q, gamma, beta):
    b, n, c = q.shape
    return pl.pallas_call(
        functools.partial(_stats_body, n=n),
        grid=(b,),
        in_specs=[
            pl.BlockSpec((1, n, K), lambda bi: (bi, 0, 0)),
            pl.BlockSpec((1, n, c), lambda bi: (bi, 0, 0)),
            pl.BlockSpec((1, c), lambda bi: (0, 0)),
            pl.BlockSpec((1, c), lambda bi: (0, 0)),
        ],
        out_specs=pl.BlockSpec((2, c), lambda bi: (0, 0)),
        out_shape=jax.ShapeDtypeStruct((2, c), jnp.float32),
        scratch_shapes=[pltpu.VMEM((2, c), jnp.float32)],
    )(idx4, q, gamma, beta)


def _affine_body(q_ref, ac_ref, o_ref):
    a = ac_ref[0:1, :]
    c = ac_ref[1:2, :]
    o_ref[0] = _leaky(q_ref[0] * a + c, 0.2)


def _affine_leaky(q, ac):
    b, n, c = q.shape
    r = 512
    return pl.pallas_call(
        _affine_body,
        grid=(b, n // r),
        in_specs=[
            pl.BlockSpec((1, r, c), lambda bi, i: (bi, i, 0)),
            pl.BlockSpec((2, c), lambda bi, i: (0, 0)),
        ],
        out_specs=pl.BlockSpec((1, r, c), lambda bi, i: (bi, i, 0)),
        out_shape=jax.ShapeDtypeStruct((b, n, c), jnp.float32),
    )(q, ac)


def _head_body(f_ref, w1_ref, b1_ref, w2_ref, b2_ref, o_ref):
    t = jnp.dot(f_ref[0], w1_ref[...], preferred_element_type=jnp.float32)
    t = _leaky(t + b1_ref[...], 0.01)
    u = jnp.dot(t, w2_ref[...], preferred_element_type=jnp.float32)
    u = u + b2_ref[...]
    o_ref[0] = 1.0 / (1.0 + jnp.exp(-u)) - 0.5


def _head(feat, w1t, b1, w2t, b2):
    b, n, c = feat.shape
    o = w2t.shape[1]
    return pl.pallas_call(
        _head_body,
        grid=(b,),
        in_specs=[
            pl.BlockSpec((1, n, c), lambda bi: (bi, 0, 0)),
            pl.BlockSpec((c, 64), lambda bi: (0, 0)),
            pl.BlockSpec((1, 64), lambda bi: (0, 0)),
            pl.BlockSpec((64, o), lambda bi: (0, 0)),
            pl.BlockSpec((1, o), lambda bi: (0, 0)),
        ],
        out_specs=pl.BlockSpec((1, n, o), lambda bi: (bi, 0, 0)),
        out_shape=jax.ShapeDtypeStruct((b, n, o), jnp.float32),
    )(feat, w1t, b1, w2t, b2)


def kernel(z, point_num, sphere, W1, b1, W2, b2, W3, b3, W4, b4, gamma, beta,
           Wx1, bx1, Wx2, bx2):
    n = sphere.shape[1]
    s = sphere[0]
    s_pad = jnp.pad(s, ((0, 0), (0, 5)))

    wst = jnp.pad(W1[:, 0:3].T, ((0, 5), (0, 0)))
    wgt = jnp.pad(W1[:, 3:6].T, ((0, 5), (0, 0)))
    wz = W1[:, 6:].T

    idx0 = _knn(s_pad[None], jnp.sum(sphere * sphere, axis=-1))
    zc = _prep1(z, wz, b1[None, :])
    out1 = _layer1(idx0[0], s_pad, wst, wgt, zc)

    idx2 = _knn(out1, jnp.sum(out1 * out1, axis=-1))
    p2 = _linear(out1, W2.T, b2[None, :], "leaky")
    out2 = _gather_mean(p2, idx2)

    idx3 = _knn(out2, jnp.sum(out2 * out2, axis=-1))
    p3 = _linear(out2, W3.T, b3[None, :], "leaky")
    out3 = _gather_mean(p3, idx3)

    idx4 = _knn(out3, jnp.sum(out3 * out3, axis=-1))
    q = _linear(out3, W4.T, b4[None, :], "none")
    ac = _stats(idx4, q, gamma[None, :], beta[None, :])
    r_mat = _affine_leaky(q, ac)
    feat_nt = _gather_mean(r_mat, idx4)

    head_out = _head(feat_nt, Wx1.T, bx1[None, :],
                     jnp.pad(Wx2.T, ((0, 0), (0, 5))),
                     jnp.pad(bx2, (0, 5))[None, :])
    pcs = head_out[:, :, :3]
    feature = jnp.swapaxes(feat_nt, 1, 2)
    return pcs, feature
